# baseline (speedup 1.0000x reference)
import functools

import jax
import jax.numpy as jnp
from jax import lax
from jax.experimental import pallas as pl
from jax.experimental.pallas import tpu as pltpu
from jax.experimental.pallas import tpu_sc as plsc

B = 16384
D = 32
NUM_TABLES = 4
V_SMALL = 1000
V_STORE = 100000
CH = 4096
NCH = B // CH
L = 16
UNROLL = 16
NBUF = 3

_info = plsc.get_sparse_core_info()
NC = _info.num_cores
NS = _info.num_subcores
NW = NC * NS


@functools.partial(
    pl.kernel,
    out_type=jax.ShapeDtypeStruct((NUM_TABLES, D, B), jnp.float32),
    mesh=plsc.VectorSubcoreMesh(core_axis_name="c", subcore_axis_name="s"),
    compiler_params=pltpu.CompilerParams(use_tc_tiling_on_sc=True,
                                         needs_layout_passes=False),
    scratch_types=(
        [pltpu.VMEM((V_STORE,), jnp.float32)]
        + [pltpu.VMEM((V_SMALL,), jnp.float32) for _ in range(3)]
        + [pltpu.VMEM((CH,), jnp.int32) for _ in range(3)]
        + [pltpu.VMEM((CH,), jnp.float32) for _ in range(3)]
        + [pltpu.SemaphoreType.DMA for _ in range(10)]
    ),
)
def _encode(cat_h, col_h, fab_h, store_h,
            cat_t, col_t, fab_t, store_t,
            out_h,
            store_row, row0, row1, row2,
            idx0, idx1, idx2, ob0, ob1, ob2,
            *sems):
    k = lax.axis_index("s") * NC + lax.axis_index("c")
    idx_srcs = (cat_h, col_h, fab_h, store_h)
    rows = (row0, row1, row2, store_row)
    idx_bufs = (idx0, idx1, idx2)
    out_bufs = (ob0, ob1, ob2)
    row_sems = sems[0:4]
    idx_sems = sems[4:7]
    out_sems = sems[7:10]

    row_cp = [
        pltpu.async_copy(cat_t.at[k], row0, row_sems[0]),
        pltpu.async_copy(col_t.at[k], row1, row_sems[1]),
        pltpu.async_copy(fab_t.at[k], row2, row_sems[2]),
        pltpu.async_copy(store_t.at[k], store_row, row_sems[3]),
    ]

    stages = [(t, c) for t in range(NUM_TABLES) for c in range(NCH)]
    idx_cp = {}
    out_cp = {}
    t0, c0 = stages[0]
    idx_cp[0] = pltpu.async_copy(
        idx_srcs[t0].at[pl.ds(c0 * CH, CH)], idx_bufs[0], idx_sems[0])

    for s, (t, c) in enumerate(stages):
      with jax.named_scope(f"stage_t{t}_c{c}"):
        if s + 1 < len(stages):
            tn, cn = stages[s + 1]
            idx_cp[s + 1] = pltpu.async_copy(
                idx_srcs[tn].at[pl.ds(cn * CH, CH)],
                idx_bufs[(s + 1) % NBUF], idx_sems[(s + 1) % NBUF])
        if c == 0:
            with jax.named_scope(f"rowwait_t{t}"):
                row_cp[t].wait()
        idx_cp.pop(s).wait()
        if s >= NBUF:
            out_cp.pop(s - NBUF).wait()
        ib = idx_bufs[s % NBUF]
        ob = out_bufs[s % NBUF]
        row = rows[t]

        def body(i, _):
            base = i * (L * UNROLL)
            for u in range(UNROLL):
                iv = ib[pl.ds(base + u * L, L)]
                ob[pl.ds(base + u * L, L)] = plsc.load_gather(row, [iv])
            return 0

        lax.fori_loop(0, CH // (L * UNROLL), body, 0)
        out_cp[s] = pltpu.async_copy(
            ob, out_h.at[t, k, pl.ds(c * CH, CH)], out_sems[s % NBUF])

    for s in sorted(out_cp):
        out_cp[s].wait()


def kernel(cat, col, fab, store, cat_table, col_table, fab_table, store_table):
    out_phys = _encode(cat, col, fab, store,
                       cat_table.T, col_table.T, fab_table.T, store_table.T)
    return jnp.transpose(out_phys, (2, 0, 1))


# speedup vs baseline: 1.0343x; 1.0343x over previous
import functools

import jax
import jax.numpy as jnp
from jax import lax
from jax.experimental import pallas as pl
from jax.experimental.pallas import tpu as pltpu
from jax.experimental.pallas import tpu_sc as plsc

B = 16384
D = 32
NUM_TABLES = 4
V_SMALL = 1000
V_STORE = 100000
CH = 4096
NCH = B // CH
L = 16
UNROLL = 8
NBUF = 3

_info = plsc.get_sparse_core_info()
NC = _info.num_cores
NS = _info.num_subcores
NW = NC * NS


@functools.partial(
    pl.kernel,
    out_type=jax.ShapeDtypeStruct((NUM_TABLES, D, B), jnp.float32),
    mesh=plsc.VectorSubcoreMesh(core_axis_name="c", subcore_axis_name="s"),
    compiler_params=pltpu.CompilerParams(use_tc_tiling_on_sc=True,
                                         needs_layout_passes=False),
    scratch_types=(
        [pltpu.VMEM((V_STORE,), jnp.float32)]
        + [pltpu.VMEM((V_SMALL,), jnp.float32) for _ in range(3)]
        + [pltpu.VMEM((CH,), jnp.int32) for _ in range(3)]
        + [pltpu.VMEM((CH,), jnp.float32) for _ in range(3)]
        + [pltpu.SemaphoreType.DMA for _ in range(10)]
    ),
)
def _encode(cat_h, col_h, fab_h, store_h,
            cat_t, col_t, fab_t, store_t,
            out_h,
            store_row, row0, row1, row2,
            idx0, idx1, idx2, ob0, ob1, ob2,
            *sems):
    k = lax.axis_index("s") * NC + lax.axis_index("c")
    idx_srcs = (cat_h, col_h, fab_h, store_h)
    rows = (row0, row1, row2, store_row)
    idx_bufs = (idx0, idx1, idx2)
    out_bufs = (ob0, ob1, ob2)
    row_sems = sems[0:4]
    idx_sems = sems[4:7]
    out_sems = sems[7:10]

    row_cp = [
        pltpu.async_copy(cat_t.at[k], row0, row_sems[0]),
        pltpu.async_copy(col_t.at[k], row1, row_sems[1]),
        pltpu.async_copy(fab_t.at[k], row2, row_sems[2]),
        pltpu.async_copy(store_t.at[k], store_row, row_sems[3]),
    ]

    stages = [(t, c) for t in range(NUM_TABLES) for c in range(NCH)]
    idx_cp = {}
    out_cp = {}
    t0, c0 = stages[0]
    idx_cp[0] = pltpu.async_copy(
        idx_srcs[t0].at[pl.ds(c0 * CH, CH)], idx_bufs[0], idx_sems[0])

    for s, (t, c) in enumerate(stages):
        if s + 1 < len(stages):
            tn, cn = stages[s + 1]
            idx_cp[s + 1] = pltpu.async_copy(
                idx_srcs[tn].at[pl.ds(cn * CH, CH)],
                idx_bufs[(s + 1) % NBUF], idx_sems[(s + 1) % NBUF])
        if c == 0:
            row_cp[t].wait()
        idx_cp.pop(s).wait()
        if s >= NBUF:
            out_cp.pop(s - NBUF).wait()
        ib = idx_bufs[s % NBUF]
        ob = out_bufs[s % NBUF]
        row = rows[t]

        def body(i, _):
            base = i * (L * UNROLL)
            for u in range(UNROLL):
                iv = ib[pl.ds(base + u * L, L)]
                ob[pl.ds(base + u * L, L)] = plsc.load_gather(row, [iv])
            return 0

        lax.fori_loop(0, CH // (L * UNROLL), body, 0)
        out_cp[s] = pltpu.async_copy(
            ob, out_h.at[t, k, pl.ds(c * CH, CH)], out_sems[s % NBUF])

    for s in sorted(out_cp):
        out_cp[s].wait()


def kernel(cat, col, fab, store, cat_table, col_table, fab_table, store_table):
    out_phys = _encode(cat, col, fab, store,
                       cat_table.T, col_table.T, fab_table.T, store_table.T)
    return jnp.transpose(out_phys, (2, 0, 1))


# speedup vs baseline: 1.2518x; 1.2103x over previous
import functools

import jax
import jax.numpy as jnp
from jax import lax
from jax.experimental import pallas as pl
from jax.experimental.pallas import tpu as pltpu
from jax.experimental.pallas import tpu_sc as plsc

B = 16384
D = 32
NUM_TABLES = 4
V_SMALL = 1000
V_STORE = 100000
CH = 4096
NCH = B // CH
L = 16
UNROLL = 8
NBUF = 2

_info = plsc.get_sparse_core_info()
NC = _info.num_cores
NS = _info.num_subcores
NW = NC * NS


@functools.partial(
    pl.kernel,
    out_type=jax.ShapeDtypeStruct((NUM_TABLES, D, B), jnp.float32),
    mesh=plsc.VectorSubcoreMesh(core_axis_name="c", subcore_axis_name="s"),
    compiler_params=pltpu.CompilerParams(use_tc_tiling_on_sc=True,
                                         needs_layout_passes=False),
    scratch_types=(
        [pltpu.VMEM((V_STORE,), jnp.float32)]
        + [pltpu.VMEM((V_SMALL,), jnp.float32) for _ in range(3)]
        + [pltpu.VMEM((CH,), jnp.int32) for _ in range(2)]
        + [pltpu.VMEM((CH,), jnp.float32) for _ in range(2)]
        + [pltpu.VMEM_SHARED((NUM_TABLES * B,), jnp.int32)]
        + [pltpu.SemaphoreType.DMA for _ in range(14)]
    ),
)
def _encode(cat_h, col_h, fab_h, store_h,
            cat_t, col_t, fab_t, store_t,
            out_h,
            store_row, row0, row1, row2,
            idx0, idx1, ob0, ob1,
            idx_sh,
            *sems):
    k = lax.axis_index("s") * NC + lax.axis_index("c")
    idx_srcs = (cat_h, col_h, fab_h, store_h)
    rows = (row0, row1, row2, store_row)
    idx_bufs = (idx0, idx1)
    out_bufs = (ob0, ob1)
    row_sems = sems[0:4]
    idx_sems = sems[4:7]
    out_sems = sems[7:10]
    stage_sems = sems[10:14]

    row_cp = [
        pltpu.async_copy(cat_t.at[k], row0, row_sems[0]),
        pltpu.async_copy(col_t.at[k], row1, row_sems[1]),
        pltpu.async_copy(fab_t.at[k], row2, row_sems[2]),
        pltpu.async_copy(store_t.at[k], store_row, row_sems[3]),
    ]

    sid = lax.axis_index("s")
    @pl.when(sid == 0)
    def _stage_indices():
        st_cp = [pltpu.async_copy(idx_srcs[t],
                                  idx_sh.at[pl.ds(t * B, B)],
                                  stage_sems[t])
                 for t in range(NUM_TABLES)]
        for cp in st_cp:
            cp.wait()
    plsc.subcore_barrier()

    stages = [(t, c) for t in range(NUM_TABLES) for c in range(NCH)]
    idx_cp = {}
    out_cp = {}
    t0, c0 = stages[0]
    idx_cp[0] = pltpu.async_copy(
        idx_sh.at[pl.ds((t0 * NCH + c0) * CH, CH)], idx_bufs[0], idx_sems[0])

    for s, (t, c) in enumerate(stages):
        if s + 1 < len(stages):
            tn, cn = stages[s + 1]
            idx_cp[s + 1] = pltpu.async_copy(
                idx_sh.at[pl.ds((tn * NCH + cn) * CH, CH)],
                idx_bufs[(s + 1) % NBUF], idx_sems[(s + 1) % NBUF])
        if c == 0:
            row_cp[t].wait()
        idx_cp.pop(s).wait()
        if s >= NBUF:
            out_cp.pop(s - NBUF).wait()
        ib = idx_bufs[s % NBUF]
        ob = out_bufs[s % NBUF]
        row = rows[t]

        def body(i, _):
            base = i * (L * UNROLL)
            for u in range(UNROLL):
                iv = ib[pl.ds(base + u * L, L)]
                ob[pl.ds(base + u * L, L)] = plsc.load_gather(row, [iv])
            return 0

        lax.fori_loop(0, CH // (L * UNROLL), body, 0)
        out_cp[s] = pltpu.async_copy(
            ob, out_h.at[t, k, pl.ds(c * CH, CH)], out_sems[s % NBUF])

    for s in sorted(out_cp):
        out_cp[s].wait()


def kernel(cat, col, fab, store, cat_table, col_table, fab_table, store_table):
    out_phys = _encode(cat, col, fab, store,
                       cat_table.T, col_table.T, fab_table.T, store_table.T)
    return jnp.transpose(out_phys, (2, 0, 1))


# speedup vs baseline: 1.3794x; 1.1019x over previous
import functools

import jax
import jax.numpy as jnp
from jax import lax
from jax.experimental import pallas as pl
from jax.experimental.pallas import tpu as pltpu
from jax.experimental.pallas import tpu_sc as plsc

B = 16384
D = 32
NUM_TABLES = 4
V_SMALL = 1000
V_STORE = 100000
CH = 2048
NCH = B // CH
L = 16
UNROLL = 8

_info = plsc.get_sparse_core_info()
NC = _info.num_cores
NS = _info.num_subcores
NW = NC * NS

STREAM_TABLES = (0, 1)
VECTOR_TABLES = (2, 3)


@functools.partial(
    pl.kernel,
    out_type=jax.ShapeDtypeStruct((NUM_TABLES, D, B), jnp.float32),
    mesh=plsc.VectorSubcoreMesh(core_axis_name="c", subcore_axis_name="s"),
    compiler_params=pltpu.CompilerParams(use_tc_tiling_on_sc=True,
                                         needs_layout_passes=False),
    scratch_types=(
        [pltpu.VMEM((V_STORE,), jnp.float32)]
        + [pltpu.VMEM((V_SMALL,), jnp.float32)]
        + [pltpu.VMEM((V_SMALL,), jnp.float32)] * 2
        + [pltpu.VMEM_SHARED((NS * V_SMALL,), jnp.float32)] * 2
        + [pltpu.VMEM_SHARED((NUM_TABLES * B,), jnp.int32)]
        + [pltpu.VMEM((CH,), jnp.int32)] * 4
        + [pltpu.VMEM((CH,), jnp.float32)] * 4
        + [pltpu.SemaphoreType.DMA] * 16
    ),
)
def _encode(cat_h, col_h, fab_h, store_h,
            cat_t, col_t, fab_t, store_t,
            out_h,
            store_row, fab_row, b0, b1, sh0, sh1, idx_sh,
            iv0, iv1, is0, is1, ov0, ov1, os0, os1,
            *sems):
    sid = lax.axis_index("s")
    k = sid * NC + lax.axis_index("c")
    idx_srcs = (cat_h, col_h, fab_h, store_h)
    row_sems = sems[0:4]
    idx_sems = sems[4:8]
    out_sems = sems[8:12]
    g_sems = sems[12:14]
    st_sems = sems[14:16]

    vec_rows = {2: fab_row, 3: store_row}
    sh_rows = {0: sh0, 1: sh1}
    bounce = {0: b0, 1: b1}
    iv_bufs = (iv0, iv1)
    is_bufs = (is0, is1)
    ov_bufs = (ov0, ov1)
    os_bufs = (os0, os1)

    row_cp = {
        0: pltpu.async_copy(cat_t.at[k], b0, row_sems[0]),
        1: pltpu.async_copy(col_t.at[k], b1, row_sems[1]),
        2: pltpu.async_copy(fab_t.at[k], fab_row, row_sems[2]),
        3: pltpu.async_copy(store_t.at[k], store_row, row_sems[3]),
    }

    @pl.when(sid == 0)
    def _stage_indices():
        st_cp = [pltpu.async_copy(idx_srcs[t],
                                  idx_sh.at[pl.ds(t * B, B)],
                                  st_sems[0])
                 for t in range(NUM_TABLES)]
        for cp in st_cp:
            cp.wait()
    plsc.subcore_barrier()

    for t in STREAM_TABLES:
        row_cp[t].wait()
        row_cp[t] = pltpu.async_copy(
            bounce[t], sh_rows[t].at[pl.ds(sid * V_SMALL, V_SMALL)],
            row_sems[t])

    def idx_slice(t, c):
        return idx_sh.at[pl.ds((t * B) + c * CH, CH)]

    pairs = [(STREAM_TABLES[c // NCH], VECTOR_TABLES[c // NCH], c % NCH)
             for c in range(2 * NCH)]

    ts0, tv0, c0 = pairs[0]
    idx_cp = {}
    out_cp = {}
    idx_cp["s0"] = pltpu.async_copy(idx_slice(ts0, c0), is_bufs[0], idx_sems[0])
    idx_cp["v0"] = pltpu.async_copy(idx_slice(tv0, c0), iv_bufs[0], idx_sems[2])

    for p, (ts, tv, c) in enumerate(pairs):
        if p + 1 < len(pairs):
            tsn, tvn, cn = pairs[p + 1]
            nb = (p + 1) % 2
            idx_cp[f"s{p + 1}"] = pltpu.async_copy(
                idx_slice(tsn, cn), is_bufs[nb], idx_sems[nb])
            idx_cp[f"v{p + 1}"] = pltpu.async_copy(
                idx_slice(tvn, cn), iv_bufs[nb], idx_sems[2 + nb])
        if c == 0:
            row_cp[ts].wait()
            row_cp[tv].wait()
        pb = p % 2
        for key in (f"s{p - 2}", f"v{p - 2}"):
            if key in out_cp:
                out_cp.pop(key).wait()

        idx_cp.pop(f"s{p}").wait()
        g_cp = pltpu.async_copy(
            sh_rows[ts].at[pl.ds(sid * V_SMALL, V_SMALL)].at[is_bufs[pb]],
            os_bufs[pb], g_sems[pb])

        idx_cp.pop(f"v{p}").wait()
        ib = iv_bufs[pb]
        ob = ov_bufs[pb]
        row = vec_rows[tv]

        def body(i, _):
            base = i * (L * UNROLL)
            for u in range(UNROLL):
                ivec = ib[pl.ds(base + u * L, L)]
                ob[pl.ds(base + u * L, L)] = plsc.load_gather(row, [ivec])
            return 0

        lax.fori_loop(0, CH // (L * UNROLL), body, 0)

        out_cp[f"v{p}"] = pltpu.async_copy(
            ob, out_h.at[tv, k, pl.ds(c * CH, CH)], out_sems[pb])
        g_cp.wait()
        out_cp[f"s{p}"] = pltpu.async_copy(
            os_bufs[pb], out_h.at[ts, k, pl.ds(c * CH, CH)], out_sems[2 + pb])

    for key in sorted(out_cp):
        out_cp.pop(key).wait()


def kernel(cat, col, fab, store, cat_table, col_table, fab_table, store_table):
    out_phys = _encode(cat, col, fab, store,
                       cat_table.T, col_table.T, fab_table.T, store_table.T)
    return jnp.transpose(out_phys, (2, 0, 1))
